# Initial kernel scaffold; baseline (speedup 1.0000x reference)
import functools

import jax
import jax.numpy as jnp
from jax import lax
from jax.experimental import pallas as pl
from jax.experimental.pallas import tpu as pltpu
from jax.experimental.pallas import tpu_sc as plsc

N = 10000
E = 320000
D_IN = 128
HEAD = 32

NUM_SC = 2
TILES = 16
CHUNK = 128
CH_PER_TILE = 157
EPT = CH_PER_TILE * CHUNK
EPAD = EPT * TILES
PAD = EPAD - E
RPT = N // TILES

BN = 1000


def _build_edges(adj0, adj1):
    pad_src = jnp.zeros((PAD,), jnp.int32)
    pad_dst = jnp.full((PAD,), N, jnp.int32)
    src = jnp.concatenate([adj0[0], pad_src, adj1[0] + N, pad_src])
    dst = jnp.concatenate([adj0[1], pad_dst, adj1[1], pad_dst])
    return (src.reshape(NUM_SC * TILES * CH_PER_TILE, CHUNK),
            dst.reshape(NUM_SC * TILES * CH_PER_TILE, CHUNK))


def _sc_scatter_body(zcat, srcm, dstm, zeros_hbm, out, srcv, dstv, rows, acc):
    cid = lax.axis_index("c")
    sid = lax.axis_index("s")

    pltpu.sync_copy(zeros_hbm.at[pl.ds(sid * RPT, RPT)],
                    acc.at[pl.ds(sid * RPT, RPT)])

    row0 = (cid * TILES + sid) * CH_PER_TILE
    pltpu.sync_copy(srcm.at[pl.ds(row0, CH_PER_TILE)], srcv)
    pltpu.sync_copy(dstm.at[pl.ds(row0, CH_PER_TILE)], dstv)

    plsc.subcore_barrier()

    def body(k, carry):
        pltpu.sync_copy(zcat.at[srcv.at[k]], rows)
        pltpu.sync_copy(rows, acc.at[dstv.at[k]], add=True)
        return carry

    lax.fori_loop(0, CH_PER_TILE, body, 0, unroll=False)

    plsc.subcore_barrier()

    pltpu.sync_copy(acc.at[pl.ds(sid * RPT, RPT)],
                    out.at[pl.ds(cid * N + sid * RPT, RPT)])


_sc_scatter = functools.partial(
    pl.kernel,
    out_type=jax.ShapeDtypeStruct((NUM_SC * N, HEAD), jnp.float32),
    mesh=plsc.VectorSubcoreMesh(core_axis_name="c", subcore_axis_name="s"),
    scratch_types=[
        pltpu.VMEM((CH_PER_TILE, CHUNK), jnp.int32),
        pltpu.VMEM((CH_PER_TILE, CHUNK), jnp.int32),
        pltpu.VMEM((CHUNK, HEAD), jnp.float32),
        pltpu.VMEM_SHARED((N + 8, HEAD), jnp.float32),
    ],
)(_sc_scatter_body)


def _mm_in_body(x_ref, w_ref, o_ref):
    z = jnp.dot(x_ref[...], w_ref[...], preferred_element_type=jnp.float32)
    o_ref[0] = z[:, :HEAD]
    o_ref[1] = z[:, HEAD:]


def _layer0_matmul(x, wcat):
    return pl.pallas_call(
        _mm_in_body,
        grid=(N // BN,),
        in_specs=[
            pl.BlockSpec((BN, D_IN), lambda i: (i, 0)),
            pl.BlockSpec((D_IN, 2 * HEAD), lambda i: (0, 0)),
        ],
        out_specs=pl.BlockSpec((2, BN, HEAD), lambda i: (0, i, 0)),
        out_shape=jax.ShapeDtypeStruct((2, N, HEAD), jnp.float32),
    )(x, wcat)


def _mid_body(agg_ref, b0_ref, b1_ref, w0_ref, w1_ref, h_ref, z_ref):
    h0 = jax.nn.relu(agg_ref[0] + b0_ref[...])
    h1 = jax.nn.relu(agg_ref[1] + b1_ref[...])
    hb = jnp.concatenate([h0, h1], axis=1)
    h_ref[...] = hb
    z_ref[0] = jnp.dot(hb, w0_ref[...], preferred_element_type=jnp.float32)
    z_ref[1] = jnp.dot(hb, w1_ref[...], preferred_element_type=jnp.float32)


def _layer_mid(agg, b00, b01, W10, W11):
    return pl.pallas_call(
        _mid_body,
        grid=(N // BN,),
        in_specs=[
            pl.BlockSpec((2, BN, HEAD), lambda i: (0, i, 0)),
            pl.BlockSpec((1, HEAD), lambda i: (0, 0)),
            pl.BlockSpec((1, HEAD), lambda i: (0, 0)),
            pl.BlockSpec((2 * HEAD, HEAD), lambda i: (0, 0)),
            pl.BlockSpec((2 * HEAD, HEAD), lambda i: (0, 0)),
        ],
        out_specs=[
            pl.BlockSpec((BN, 2 * HEAD), lambda i: (i, 0)),
            pl.BlockSpec((2, BN, HEAD), lambda i: (0, i, 0)),
        ],
        out_shape=[
            jax.ShapeDtypeStruct((N, 2 * HEAD), jnp.float32),
            jax.ShapeDtypeStruct((2, N, HEAD), jnp.float32),
        ],
    )(agg, b00, b01, W10, W11)


def _final_body(agg_ref, b0_ref, b1_ref, h_ref):
    h0 = jax.nn.relu(agg_ref[0] + b0_ref[...])
    h1 = jax.nn.relu(agg_ref[1] + b1_ref[...])
    h_ref[...] = jnp.concatenate([h0, h1], axis=1)


def _layer_final(agg, b10, b11):
    return pl.pallas_call(
        _final_body,
        grid=(N // BN,),
        in_specs=[
            pl.BlockSpec((2, BN, HEAD), lambda i: (0, i, 0)),
            pl.BlockSpec((1, HEAD), lambda i: (0, 0)),
            pl.BlockSpec((1, HEAD), lambda i: (0, 0)),
        ],
        out_specs=pl.BlockSpec((BN, 2 * HEAD), lambda i: (i, 0)),
        out_shape=jax.ShapeDtypeStruct((N, 2 * HEAD), jnp.float32),
    )(agg, b10, b11)


def kernel(x, adj0, adj1, W00, b00, W01, b01, W10, b10, W11, b11):
    srcm, dstm = _build_edges(adj0, adj1)
    zeros = jnp.zeros((N, HEAD), jnp.float32)

    z0 = _layer0_matmul(x, jnp.concatenate([W00, W01], axis=1))
    agg0 = _sc_scatter(z0.reshape(NUM_SC * N, HEAD), srcm, dstm, zeros)
    h1, z1 = _layer_mid(agg0.reshape(NUM_SC, N, HEAD),
                        b00.reshape(1, HEAD), b01.reshape(1, HEAD), W10, W11)
    agg1 = _sc_scatter(z1.reshape(NUM_SC * N, HEAD), srcm, dstm, zeros)
    h2 = _layer_final(agg1.reshape(NUM_SC, N, HEAD),
                      b10.reshape(1, HEAD), b11.reshape(1, HEAD))
    return jnp.concatenate([h1, h2], axis=-1)


# speedup vs baseline: 8.1352x; 8.1352x over previous
import functools

import jax
import jax.numpy as jnp
from jax import lax
from jax.experimental import pallas as pl
from jax.experimental.pallas import tpu as pltpu
from jax.experimental.pallas import tpu_sc as plsc

N = 10000
E = 320000
D_IN = 128
HEAD = 32

NUM_SC = 2
TILES = 16
CHUNK = 128
CH_PER_TILE = 160
EPT = CH_PER_TILE * CHUNK
EPAD = EPT * TILES
PAD = EPAD - E
N_PAD = 10240
RPT = N_PAD // TILES

BN = 1000


def _build_edges(adj0, adj1):
    pad_src = jnp.zeros((PAD,), jnp.int32)
    pad_dst = jnp.full((PAD,), N, jnp.int32)
    src = jnp.concatenate([adj0[0], pad_src, adj1[0] + N, pad_src])
    dst = jnp.concatenate([adj0[1], pad_dst, adj1[1], pad_dst])
    return (src.reshape(NUM_SC * TILES * CH_PER_TILE, CHUNK),
            dst.reshape(NUM_SC * TILES * CH_PER_TILE, CHUNK))


def _sc_scatter_body(zcat, srcm, dstm, zeros_hbm, out, srcv, dstv, rows, acc):
    cid = lax.axis_index("c")
    sid = lax.axis_index("s")

    pltpu.sync_copy(zeros_hbm.at[pl.ds(sid * RPT, RPT)],
                    acc.at[pl.ds(sid * RPT, RPT)])

    row0 = (cid * TILES + sid) * CH_PER_TILE
    pltpu.sync_copy(srcm.at[pl.ds(row0, CH_PER_TILE)], srcv)
    pltpu.sync_copy(dstm.at[pl.ds(row0, CH_PER_TILE)], dstv)

    plsc.subcore_barrier()

    def body(k, carry):
        pltpu.sync_copy(zcat.at[srcv.at[k]], rows)
        pltpu.sync_copy(rows, acc.at[dstv.at[k]], add=True)
        return carry

    lax.fori_loop(0, CH_PER_TILE, body, 0, unroll=False)

    plsc.subcore_barrier()

    pltpu.sync_copy(acc.at[pl.ds(sid * RPT, RPT)],
                    out.at[pl.ds(cid * N_PAD + sid * RPT, RPT)])


_sc_scatter = functools.partial(
    pl.kernel,
    out_type=jax.ShapeDtypeStruct((NUM_SC * N_PAD, HEAD), jnp.float32),
    mesh=plsc.VectorSubcoreMesh(core_axis_name="c", subcore_axis_name="s"),
    scratch_types=[
        pltpu.VMEM((CH_PER_TILE, CHUNK), jnp.int32),
        pltpu.VMEM((CH_PER_TILE, CHUNK), jnp.int32),
        pltpu.VMEM((CHUNK, HEAD), jnp.float32),
        pltpu.VMEM_SHARED((N_PAD, HEAD), jnp.float32),
    ],
    compiler_params=pltpu.CompilerParams(use_tc_tiling_on_sc=False),
)(_sc_scatter_body)


def _mm_in_body(x_ref, w_ref, o_ref):
    z = jnp.dot(x_ref[...], w_ref[...], preferred_element_type=jnp.float32)
    o_ref[0] = z[:, :HEAD]
    o_ref[1] = z[:, HEAD:]


def _layer0_matmul(x, wcat):
    return pl.pallas_call(
        _mm_in_body,
        grid=(N // BN,),
        in_specs=[
            pl.BlockSpec((BN, D_IN), lambda i: (i, 0)),
            pl.BlockSpec((D_IN, 2 * HEAD), lambda i: (0, 0)),
        ],
        out_specs=pl.BlockSpec((2, BN, HEAD), lambda i: (0, i, 0)),
        out_shape=jax.ShapeDtypeStruct((2, N, HEAD), jnp.float32),
    )(x, wcat)


def _mid_body(agg_ref, b0_ref, b1_ref, w0_ref, w1_ref, h_ref, z_ref):
    h0 = jax.nn.relu(agg_ref[0] + b0_ref[...])
    h1 = jax.nn.relu(agg_ref[1] + b1_ref[...])
    hb = jnp.concatenate([h0, h1], axis=1)
    h_ref[...] = hb
    z_ref[0] = jnp.dot(hb, w0_ref[...], preferred_element_type=jnp.float32)
    z_ref[1] = jnp.dot(hb, w1_ref[...], preferred_element_type=jnp.float32)


def _layer_mid(agg, b00, b01, W10, W11):
    return pl.pallas_call(
        _mid_body,
        grid=(N // BN,),
        in_specs=[
            pl.BlockSpec((2, BN, HEAD), lambda i: (0, i, 0)),
            pl.BlockSpec((1, HEAD), lambda i: (0, 0)),
            pl.BlockSpec((1, HEAD), lambda i: (0, 0)),
            pl.BlockSpec((2 * HEAD, HEAD), lambda i: (0, 0)),
            pl.BlockSpec((2 * HEAD, HEAD), lambda i: (0, 0)),
        ],
        out_specs=[
            pl.BlockSpec((BN, 2 * HEAD), lambda i: (i, 0)),
            pl.BlockSpec((2, BN, HEAD), lambda i: (0, i, 0)),
        ],
        out_shape=[
            jax.ShapeDtypeStruct((N, 2 * HEAD), jnp.float32),
            jax.ShapeDtypeStruct((2, N, HEAD), jnp.float32),
        ],
    )(agg, b00, b01, W10, W11)


def _final_body(agg_ref, b0_ref, b1_ref, h_ref):
    h0 = jax.nn.relu(agg_ref[0] + b0_ref[...])
    h1 = jax.nn.relu(agg_ref[1] + b1_ref[...])
    h_ref[...] = jnp.concatenate([h0, h1], axis=1)


def _layer_final(agg, b10, b11):
    return pl.pallas_call(
        _final_body,
        grid=(N // BN,),
        in_specs=[
            pl.BlockSpec((2, BN, HEAD), lambda i: (0, i, 0)),
            pl.BlockSpec((1, HEAD), lambda i: (0, 0)),
            pl.BlockSpec((1, HEAD), lambda i: (0, 0)),
        ],
        out_specs=pl.BlockSpec((BN, 2 * HEAD), lambda i: (i, 0)),
        out_shape=jax.ShapeDtypeStruct((N, 2 * HEAD), jnp.float32),
    )(agg, b10, b11)


def kernel(x, adj0, adj1, W00, b00, W01, b01, W10, b10, W11, b11):
    srcm, dstm = _build_edges(adj0, adj1)
    zeros = jnp.zeros((N_PAD, HEAD), jnp.float32)

    z0 = _layer0_matmul(x, jnp.concatenate([W00, W01], axis=1))
    agg0 = _sc_scatter(z0.reshape(NUM_SC * N, HEAD), srcm, dstm, zeros)
    h1, z1 = _layer_mid(agg0.reshape(NUM_SC, N_PAD, HEAD),
                        b00.reshape(1, HEAD), b01.reshape(1, HEAD), W10, W11)
    agg1 = _sc_scatter(z1.reshape(NUM_SC * N, HEAD), srcm, dstm, zeros)
    h2 = _layer_final(agg1.reshape(NUM_SC, N_PAD, HEAD),
                      b10.reshape(1, HEAD), b11.reshape(1, HEAD))
    return jnp.concatenate([h1, h2], axis=-1)


# speedup vs baseline: 9.2499x; 1.1370x over previous
import functools

import jax
import jax.numpy as jnp
from jax import lax
from jax.experimental import pallas as pl
from jax.experimental.pallas import tpu as pltpu
from jax.experimental.pallas import tpu_sc as plsc

N = 10000
E = 320000
D_IN = 128
HEAD = 32

NUM_SC = 2
TILES = 16
CHUNK = 128
CH_PER_TILE = 160
EPT = CH_PER_TILE * CHUNK
EPAD = EPT * TILES
PAD = EPAD - E
N_PAD = 10240
RPT = N_PAD // TILES

BN = 1000


def _build_edges(adj0, adj1):
    pad_src = jnp.zeros((PAD,), jnp.int32)
    pad_dst = jnp.full((PAD,), N, jnp.int32)
    src = jnp.concatenate([adj0[0], pad_src, adj1[0] + N, pad_src])
    dst = jnp.concatenate([adj0[1], pad_dst, adj1[1], pad_dst])
    return (src.reshape(NUM_SC * TILES * CH_PER_TILE, CHUNK),
            dst.reshape(NUM_SC * TILES * CH_PER_TILE, CHUNK))


def _sc_scatter_body(zcat, srcm, dstm, zeros_hbm, out, srcv, dstv,
                     rows0, rows1, acc, gsem0, gsem1, ssem0, ssem1):
    cid = lax.axis_index("c")
    sid = lax.axis_index("s")

    pltpu.sync_copy(zeros_hbm.at[pl.ds(sid * RPT, RPT)],
                    acc.at[pl.ds(sid * RPT, RPT)])

    row0 = (cid * TILES + sid) * CH_PER_TILE
    pltpu.sync_copy(srcm.at[pl.ds(row0, CH_PER_TILE)], srcv)
    pltpu.sync_copy(dstm.at[pl.ds(row0, CH_PER_TILE)], dstv)

    plsc.subcore_barrier()

    def gather_start(k, buf, sem):
        pltpu.async_copy(zcat.at[srcv.at[k]], buf, sem)

    def gather_wait(k, buf, sem):
        pltpu.make_async_copy(zcat.at[srcv.at[k]], buf, sem).wait()

    def scatter_start(k, buf, sem):
        pltpu.async_copy(buf, acc.at[dstv.at[k]], sem, add=True)

    def scatter_wait(k, buf, sem):
        pltpu.make_async_copy(buf, acc.at[dstv.at[k]], sem).wait()

    gather_start(0, rows0, gsem0)
    gather_start(1, rows1, gsem1)

    def body(k2, carry):
        gather_wait(k2, rows0, gsem0)
        scatter_start(k2, rows0, ssem0)
        gather_wait(k2 + 1, rows1, gsem1)
        scatter_start(k2 + 1, rows1, ssem1)

        @pl.when(k2 + 2 < CH_PER_TILE)
        def _():
            scatter_wait(k2, rows0, ssem0)
            gather_start(k2 + 2, rows0, gsem0)
            scatter_wait(k2 + 1, rows1, ssem1)
            gather_start(k2 + 3, rows1, gsem1)

        return carry

    lax.fori_loop(0, CH_PER_TILE // 2, lambda i, c: body(2 * i, c), 0,
                  unroll=False)
    scatter_wait(CH_PER_TILE - 2, rows0, ssem0)
    scatter_wait(CH_PER_TILE - 1, rows1, ssem1)

    plsc.subcore_barrier()

    pltpu.sync_copy(acc.at[pl.ds(sid * RPT, RPT)],
                    out.at[pl.ds(cid * N_PAD + sid * RPT, RPT)])


_sc_scatter = functools.partial(
    pl.kernel,
    out_type=jax.ShapeDtypeStruct((NUM_SC * N_PAD, HEAD), jnp.float32),
    mesh=plsc.VectorSubcoreMesh(core_axis_name="c", subcore_axis_name="s"),
    scratch_types=[
        pltpu.VMEM((CH_PER_TILE, CHUNK), jnp.int32),
        pltpu.VMEM((CH_PER_TILE, CHUNK), jnp.int32),
        pltpu.VMEM((CHUNK, HEAD), jnp.float32),
        pltpu.VMEM((CHUNK, HEAD), jnp.float32),
        pltpu.VMEM_SHARED((N_PAD, HEAD), jnp.float32),
        pltpu.SemaphoreType.DMA,
        pltpu.SemaphoreType.DMA,
        pltpu.SemaphoreType.DMA,
        pltpu.SemaphoreType.DMA,
    ],
    compiler_params=pltpu.CompilerParams(use_tc_tiling_on_sc=False),
)(_sc_scatter_body)


def _mm_in_body(x_ref, w_ref, o_ref):
    z = jnp.dot(x_ref[...], w_ref[...], preferred_element_type=jnp.float32)
    o_ref[0] = z[:, :HEAD]
    o_ref[1] = z[:, HEAD:]


def _layer0_matmul(x, wcat):
    return pl.pallas_call(
        _mm_in_body,
        grid=(N // BN,),
        in_specs=[
            pl.BlockSpec((BN, D_IN), lambda i: (i, 0)),
            pl.BlockSpec((D_IN, 2 * HEAD), lambda i: (0, 0)),
        ],
        out_specs=pl.BlockSpec((2, BN, HEAD), lambda i: (0, i, 0)),
        out_shape=jax.ShapeDtypeStruct((2, N, HEAD), jnp.float32),
    )(x, wcat)


def _mid_body(agg_ref, b0_ref, b1_ref, w0_ref, w1_ref, h_ref, z_ref):
    h0 = jax.nn.relu(agg_ref[0] + b0_ref[...])
    h1 = jax.nn.relu(agg_ref[1] + b1_ref[...])
    hb = jnp.concatenate([h0, h1], axis=1)
    h_ref[...] = hb
    z_ref[0] = jnp.dot(hb, w0_ref[...], preferred_element_type=jnp.float32)
    z_ref[1] = jnp.dot(hb, w1_ref[...], preferred_element_type=jnp.float32)


def _layer_mid(agg, b00, b01, W10, W11):
    return pl.pallas_call(
        _mid_body,
        grid=(N // BN,),
        in_specs=[
            pl.BlockSpec((2, BN, HEAD), lambda i: (0, i, 0)),
            pl.BlockSpec((1, HEAD), lambda i: (0, 0)),
            pl.BlockSpec((1, HEAD), lambda i: (0, 0)),
            pl.BlockSpec((2 * HEAD, HEAD), lambda i: (0, 0)),
            pl.BlockSpec((2 * HEAD, HEAD), lambda i: (0, 0)),
        ],
        out_specs=[
            pl.BlockSpec((BN, 2 * HEAD), lambda i: (i, 0)),
            pl.BlockSpec((2, BN, HEAD), lambda i: (0, i, 0)),
        ],
        out_shape=[
            jax.ShapeDtypeStruct((N, 2 * HEAD), jnp.float32),
            jax.ShapeDtypeStruct((2, N, HEAD), jnp.float32),
        ],
    )(agg, b00, b01, W10, W11)


def _final_body(agg_ref, b0_ref, b1_ref, h_ref):
    h0 = jax.nn.relu(agg_ref[0] + b0_ref[...])
    h1 = jax.nn.relu(agg_ref[1] + b1_ref[...])
    h_ref[...] = jnp.concatenate([h0, h1], axis=1)


def _layer_final(agg, b10, b11):
    return pl.pallas_call(
        _final_body,
        grid=(N // BN,),
        in_specs=[
            pl.BlockSpec((2, BN, HEAD), lambda i: (0, i, 0)),
            pl.BlockSpec((1, HEAD), lambda i: (0, 0)),
            pl.BlockSpec((1, HEAD), lambda i: (0, 0)),
        ],
        out_specs=pl.BlockSpec((BN, 2 * HEAD), lambda i: (i, 0)),
        out_shape=jax.ShapeDtypeStruct((N, 2 * HEAD), jnp.float32),
    )(agg, b10, b11)


def kernel(x, adj0, adj1, W00, b00, W01, b01, W10, b10, W11, b11):
    srcm, dstm = _build_edges(adj0, adj1)
    zeros = jnp.zeros((N_PAD, HEAD), jnp.float32)

    z0 = _layer0_matmul(x, jnp.concatenate([W00, W01], axis=1))
    agg0 = _sc_scatter(z0.reshape(NUM_SC * N, HEAD), srcm, dstm, zeros)
    h1, z1 = _layer_mid(agg0.reshape(NUM_SC, N_PAD, HEAD),
                        b00.reshape(1, HEAD), b01.reshape(1, HEAD), W10, W11)
    agg1 = _sc_scatter(z1.reshape(NUM_SC * N, HEAD), srcm, dstm, zeros)
    h2 = _layer_final(agg1.reshape(NUM_SC, N_PAD, HEAD),
                      b10.reshape(1, HEAD), b11.reshape(1, HEAD))
    return jnp.concatenate([h1, h2], axis=-1)


# speedup vs baseline: 9.9555x; 1.0763x over previous
import functools

import jax
import jax.numpy as jnp
from jax import lax
from jax.experimental import pallas as pl
from jax.experimental.pallas import tpu as pltpu
from jax.experimental.pallas import tpu_sc as plsc

N = 10000
E = 320000
D_IN = 128
HEAD = 32

NUM_SC = 2
TILES = 16
CHUNK = 512
NOPS = 40
CH_PER_TILE = NOPS
EPT = CH_PER_TILE * CHUNK
EPAD = EPT * TILES
PAD = EPAD - E
N_PAD = 10240
RPT = N_PAD // TILES

BN = 1000


def _build_edges(adj0, adj1):
    pad_src = jnp.zeros((PAD,), jnp.int32)
    pad_dst = jnp.full((PAD,), N, jnp.int32)
    src = jnp.concatenate([adj0[0], pad_src, adj1[0] + N, pad_src])
    dst = jnp.concatenate([adj0[1], pad_dst, adj1[1], pad_dst])
    return (src.reshape(NUM_SC * TILES * NOPS, CHUNK),
            dst.reshape(NUM_SC * TILES * NOPS, CHUNK))


def _sc_scatter_body(zcat, srcm, dstm, zeros_hbm, out, srcg, dstg,
                     rows0, rows1, acc, gsem0, gsem1, ssem0, ssem1):
    cid = lax.axis_index("c")
    sid = lax.axis_index("s")

    pltpu.sync_copy(zeros_hbm.at[pl.ds(sid * RPT, RPT)],
                    acc.at[pl.ds(sid * RPT, RPT)])

    row0 = (cid * TILES + sid) * NOPS
    pltpu.sync_copy(srcm.at[pl.ds(row0, NOPS)], srcg)
    pltpu.sync_copy(dstm.at[pl.ds(row0, NOPS)], dstg)

    plsc.subcore_barrier()

    def gather_start(k, buf, sem):
        pltpu.async_copy(zcat.at[srcg.at[k]], buf, sem)

    def gather_wait(k, buf, sem):
        pltpu.make_async_copy(zcat.at[srcg.at[k]], buf, sem).wait()

    def scatter_start(k, buf, sem):
        pltpu.async_copy(buf, acc.at[dstg.at[k]], sem, add=True)

    def scatter_wait(k, buf, sem):
        pltpu.make_async_copy(buf, acc.at[dstg.at[k]], sem).wait()

    gather_start(0, rows0, gsem0)
    gather_start(1, rows1, gsem1)

    def body(k2, carry):
        gather_wait(k2, rows0, gsem0)
        scatter_start(k2, rows0, ssem0)
        gather_wait(k2 + 1, rows1, gsem1)
        scatter_start(k2 + 1, rows1, ssem1)

        @pl.when(k2 + 2 < NOPS)
        def _():
            scatter_wait(k2, rows0, ssem0)
            gather_start(k2 + 2, rows0, gsem0)
            scatter_wait(k2 + 1, rows1, ssem1)
            gather_start(k2 + 3, rows1, gsem1)

        return carry

    lax.fori_loop(0, NOPS // 2, lambda i, c: body(2 * i, c), 0,
                  unroll=False)
    scatter_wait(NOPS - 2, rows0, ssem0)
    scatter_wait(NOPS - 1, rows1, ssem1)

    plsc.subcore_barrier()

    pltpu.sync_copy(acc.at[pl.ds(sid * RPT, RPT)],
                    out.at[pl.ds(cid * N_PAD + sid * RPT, RPT)])


_sc_scatter = functools.partial(
    pl.kernel,
    out_type=jax.ShapeDtypeStruct((NUM_SC * N_PAD, HEAD), jnp.float32),
    mesh=plsc.VectorSubcoreMesh(core_axis_name="c", subcore_axis_name="s"),
    scratch_types=[
        pltpu.VMEM((NOPS, CHUNK), jnp.int32),
        pltpu.VMEM((NOPS, CHUNK), jnp.int32),
        pltpu.VMEM((CHUNK, HEAD), jnp.float32),
        pltpu.VMEM((CHUNK, HEAD), jnp.float32),
        pltpu.VMEM_SHARED((N_PAD, HEAD), jnp.float32),
        pltpu.SemaphoreType.DMA,
        pltpu.SemaphoreType.DMA,
        pltpu.SemaphoreType.DMA,
        pltpu.SemaphoreType.DMA,
    ],
    compiler_params=pltpu.CompilerParams(use_tc_tiling_on_sc=False),
)(_sc_scatter_body)


def _mm_in_body(x_ref, w_ref, o_ref):
    z = jnp.dot(x_ref[...], w_ref[...], preferred_element_type=jnp.float32)
    o_ref[0] = z[:, :HEAD]
    o_ref[1] = z[:, HEAD:]


def _layer0_matmul(x, wcat):
    return pl.pallas_call(
        _mm_in_body,
        grid=(N // BN,),
        in_specs=[
            pl.BlockSpec((BN, D_IN), lambda i: (i, 0)),
            pl.BlockSpec((D_IN, 2 * HEAD), lambda i: (0, 0)),
        ],
        out_specs=pl.BlockSpec((2, BN, HEAD), lambda i: (0, i, 0)),
        out_shape=jax.ShapeDtypeStruct((2, N, HEAD), jnp.float32),
    )(x, wcat)


def _mid_body(agg_ref, b0_ref, b1_ref, w0_ref, w1_ref, h_ref, z_ref):
    h0 = jax.nn.relu(agg_ref[0] + b0_ref[...])
    h1 = jax.nn.relu(agg_ref[1] + b1_ref[...])
    hb = jnp.concatenate([h0, h1], axis=1)
    h_ref[...] = hb
    z_ref[0] = jnp.dot(hb, w0_ref[...], preferred_element_type=jnp.float32)
    z_ref[1] = jnp.dot(hb, w1_ref[...], preferred_element_type=jnp.float32)


def _layer_mid(agg, b00, b01, W10, W11):
    return pl.pallas_call(
        _mid_body,
        grid=(N // BN,),
        in_specs=[
            pl.BlockSpec((2, BN, HEAD), lambda i: (0, i, 0)),
            pl.BlockSpec((1, HEAD), lambda i: (0, 0)),
            pl.BlockSpec((1, HEAD), lambda i: (0, 0)),
            pl.BlockSpec((2 * HEAD, HEAD), lambda i: (0, 0)),
            pl.BlockSpec((2 * HEAD, HEAD), lambda i: (0, 0)),
        ],
        out_specs=[
            pl.BlockSpec((BN, 2 * HEAD), lambda i: (i, 0)),
            pl.BlockSpec((2, BN, HEAD), lambda i: (0, i, 0)),
        ],
        out_shape=[
            jax.ShapeDtypeStruct((N, 2 * HEAD), jnp.float32),
            jax.ShapeDtypeStruct((2, N, HEAD), jnp.float32),
        ],
    )(agg, b00, b01, W10, W11)


def _final_body(agg_ref, b0_ref, b1_ref, h_ref):
    h0 = jax.nn.relu(agg_ref[0] + b0_ref[...])
    h1 = jax.nn.relu(agg_ref[1] + b1_ref[...])
    h_ref[...] = jnp.concatenate([h0, h1], axis=1)


def _layer_final(agg, b10, b11):
    return pl.pallas_call(
        _final_body,
        grid=(N // BN,),
        in_specs=[
            pl.BlockSpec((2, BN, HEAD), lambda i: (0, i, 0)),
            pl.BlockSpec((1, HEAD), lambda i: (0, 0)),
            pl.BlockSpec((1, HEAD), lambda i: (0, 0)),
        ],
        out_specs=pl.BlockSpec((BN, 2 * HEAD), lambda i: (i, 0)),
        out_shape=jax.ShapeDtypeStruct((N, 2 * HEAD), jnp.float32),
    )(agg, b10, b11)


def kernel(x, adj0, adj1, W00, b00, W01, b01, W10, b10, W11, b11):
    srcm, dstm = _build_edges(adj0, adj1)
    zeros = jnp.zeros((N_PAD, HEAD), jnp.float32)

    z0 = _layer0_matmul(x, jnp.concatenate([W00, W01], axis=1))
    agg0 = _sc_scatter(z0.reshape(NUM_SC * N, HEAD), srcm, dstm, zeros)
    h1, z1 = _layer_mid(agg0.reshape(NUM_SC, N_PAD, HEAD),
                        b00.reshape(1, HEAD), b01.reshape(1, HEAD), W10, W11)
    agg1 = _sc_scatter(z1.reshape(NUM_SC * N, HEAD), srcm, dstm, zeros)
    h2 = _layer_final(agg1.reshape(NUM_SC, N_PAD, HEAD),
                      b10.reshape(1, HEAD), b11.reshape(1, HEAD))
    return jnp.concatenate([h1, h2], axis=-1)


# speedup vs baseline: 9.9883x; 1.0033x over previous
import functools

import jax
import jax.numpy as jnp
from jax import lax
from jax.experimental import pallas as pl
from jax.experimental.pallas import tpu as pltpu
from jax.experimental.pallas import tpu_sc as plsc

N = 10000
E = 320000
D_IN = 128
HEAD = 32

NUM_SC = 2
TILES = 16
CHUNK = 1024
NOPS = 20
CH_PER_TILE = NOPS
EPT = CH_PER_TILE * CHUNK
EPAD = EPT * TILES
PAD = EPAD - E
N_PAD = 10240
RPT = N_PAD // TILES

BN = 1000


def _build_edges(adj0, adj1):
    pad_src = jnp.zeros((PAD,), jnp.int32)
    pad_dst = jnp.full((PAD,), N, jnp.int32)
    src = jnp.concatenate([adj0[0], pad_src, adj1[0] + N, pad_src])
    dst = jnp.concatenate([adj0[1], pad_dst, adj1[1], pad_dst])
    return (src.reshape(NUM_SC * TILES * NOPS, CHUNK),
            dst.reshape(NUM_SC * TILES * NOPS, CHUNK))


def _sc_scatter_body(zcat, srcm, dstm, zeros_hbm, out, srcg, dstg,
                     rows0, rows1, acc, gsem0, gsem1, ssem0, ssem1):
    cid = lax.axis_index("c")
    sid = lax.axis_index("s")

    pltpu.sync_copy(zeros_hbm.at[pl.ds(sid * RPT, RPT)],
                    acc.at[pl.ds(sid * RPT, RPT)])

    row0 = (cid * TILES + sid) * NOPS
    pltpu.sync_copy(srcm.at[pl.ds(row0, NOPS)], srcg)
    pltpu.sync_copy(dstm.at[pl.ds(row0, NOPS)], dstg)

    plsc.subcore_barrier()

    def gather_start(k, buf, sem):
        pltpu.async_copy(zcat.at[srcg.at[k]], buf, sem)

    def gather_wait(k, buf, sem):
        pltpu.make_async_copy(zcat.at[srcg.at[k]], buf, sem).wait()

    def scatter_start(k, buf, sem):
        pltpu.async_copy(buf, acc.at[dstg.at[k]], sem, add=True)

    def scatter_wait(k, buf, sem):
        pltpu.make_async_copy(buf, acc.at[dstg.at[k]], sem).wait()

    gather_start(0, rows0, gsem0)
    gather_start(1, rows1, gsem1)

    def body(k2, carry):
        gather_wait(k2, rows0, gsem0)
        scatter_start(k2, rows0, ssem0)
        gather_wait(k2 + 1, rows1, gsem1)
        scatter_start(k2 + 1, rows1, ssem1)

        @pl.when(k2 + 2 < NOPS)
        def _():
            scatter_wait(k2, rows0, ssem0)
            gather_start(k2 + 2, rows0, gsem0)
            scatter_wait(k2 + 1, rows1, ssem1)
            gather_start(k2 + 3, rows1, gsem1)

        return carry

    lax.fori_loop(0, NOPS // 2, lambda i, c: body(2 * i, c), 0,
                  unroll=False)
    scatter_wait(NOPS - 2, rows0, ssem0)
    scatter_wait(NOPS - 1, rows1, ssem1)

    plsc.subcore_barrier()

    pltpu.sync_copy(acc.at[pl.ds(sid * RPT, RPT)],
                    out.at[pl.ds(cid * N_PAD + sid * RPT, RPT)])


_sc_scatter = functools.partial(
    pl.kernel,
    out_type=jax.ShapeDtypeStruct((NUM_SC * N_PAD, HEAD), jnp.float32),
    mesh=plsc.VectorSubcoreMesh(core_axis_name="c", subcore_axis_name="s"),
    scratch_types=[
        pltpu.VMEM((NOPS, CHUNK), jnp.int32),
        pltpu.VMEM((NOPS, CHUNK), jnp.int32),
        pltpu.VMEM((CHUNK, HEAD), jnp.float32),
        pltpu.VMEM((CHUNK, HEAD), jnp.float32),
        pltpu.VMEM_SHARED((N_PAD, HEAD), jnp.float32),
        pltpu.SemaphoreType.DMA,
        pltpu.SemaphoreType.DMA,
        pltpu.SemaphoreType.DMA,
        pltpu.SemaphoreType.DMA,
    ],
    compiler_params=pltpu.CompilerParams(use_tc_tiling_on_sc=False),
)(_sc_scatter_body)


def _mm_in_body(x_ref, w_ref, o_ref):
    z = jnp.dot(x_ref[...], w_ref[...], preferred_element_type=jnp.float32)
    o_ref[0] = z[:, :HEAD]
    o_ref[1] = z[:, HEAD:]


def _layer0_matmul(x, wcat):
    return pl.pallas_call(
        _mm_in_body,
        grid=(N // BN,),
        in_specs=[
            pl.BlockSpec((BN, D_IN), lambda i: (i, 0)),
            pl.BlockSpec((D_IN, 2 * HEAD), lambda i: (0, 0)),
        ],
        out_specs=pl.BlockSpec((2, BN, HEAD), lambda i: (0, i, 0)),
        out_shape=jax.ShapeDtypeStruct((2, N, HEAD), jnp.float32),
    )(x, wcat)


def _mid_body(agg_ref, b0_ref, b1_ref, w0_ref, w1_ref, h_ref, z_ref):
    h0 = jax.nn.relu(agg_ref[0] + b0_ref[...])
    h1 = jax.nn.relu(agg_ref[1] + b1_ref[...])
    hb = jnp.concatenate([h0, h1], axis=1)
    h_ref[...] = hb
    z_ref[0] = jnp.dot(hb, w0_ref[...], preferred_element_type=jnp.float32)
    z_ref[1] = jnp.dot(hb, w1_ref[...], preferred_element_type=jnp.float32)


def _layer_mid(agg, b00, b01, W10, W11):
    return pl.pallas_call(
        _mid_body,
        grid=(N // BN,),
        in_specs=[
            pl.BlockSpec((2, BN, HEAD), lambda i: (0, i, 0)),
            pl.BlockSpec((1, HEAD), lambda i: (0, 0)),
            pl.BlockSpec((1, HEAD), lambda i: (0, 0)),
            pl.BlockSpec((2 * HEAD, HEAD), lambda i: (0, 0)),
            pl.BlockSpec((2 * HEAD, HEAD), lambda i: (0, 0)),
        ],
        out_specs=[
            pl.BlockSpec((BN, 2 * HEAD), lambda i: (i, 0)),
            pl.BlockSpec((2, BN, HEAD), lambda i: (0, i, 0)),
        ],
        out_shape=[
            jax.ShapeDtypeStruct((N, 2 * HEAD), jnp.float32),
            jax.ShapeDtypeStruct((2, N, HEAD), jnp.float32),
        ],
    )(agg, b00, b01, W10, W11)


def _final_body(agg_ref, b0_ref, b1_ref, h_ref):
    h0 = jax.nn.relu(agg_ref[0] + b0_ref[...])
    h1 = jax.nn.relu(agg_ref[1] + b1_ref[...])
    h_ref[...] = jnp.concatenate([h0, h1], axis=1)


def _layer_final(agg, b10, b11):
    return pl.pallas_call(
        _final_body,
        grid=(N // BN,),
        in_specs=[
            pl.BlockSpec((2, BN, HEAD), lambda i: (0, i, 0)),
            pl.BlockSpec((1, HEAD), lambda i: (0, 0)),
            pl.BlockSpec((1, HEAD), lambda i: (0, 0)),
        ],
        out_specs=pl.BlockSpec((BN, 2 * HEAD), lambda i: (i, 0)),
        out_shape=jax.ShapeDtypeStruct((N, 2 * HEAD), jnp.float32),
    )(agg, b10, b11)


def kernel(x, adj0, adj1, W00, b00, W01, b01, W10, b10, W11, b11):
    srcm, dstm = _build_edges(adj0, adj1)
    zeros = jnp.zeros((N_PAD, HEAD), jnp.float32)

    z0 = _layer0_matmul(x, jnp.concatenate([W00, W01], axis=1))
    agg0 = _sc_scatter(z0.reshape(NUM_SC * N, HEAD), srcm, dstm, zeros)
    h1, z1 = _layer_mid(agg0.reshape(NUM_SC, N_PAD, HEAD),
                        b00.reshape(1, HEAD), b01.reshape(1, HEAD), W10, W11)
    agg1 = _sc_scatter(z1.reshape(NUM_SC * N, HEAD), srcm, dstm, zeros)
    h2 = _layer_final(agg1.reshape(NUM_SC, N_PAD, HEAD),
                      b10.reshape(1, HEAD), b11.reshape(1, HEAD))
    return jnp.concatenate([h1, h2], axis=-1)


# speedup vs baseline: 22.4005x; 2.2427x over previous
import functools

import jax
import jax.numpy as jnp
from jax import lax
from jax.experimental import pallas as pl
from jax.experimental.pallas import tpu as pltpu
from jax.experimental.pallas import tpu_sc as plsc

N = 10000
E = 320000
D_IN = 128
HEAD = 32

NUM_SC = 2
TILES = 16
CHUNK = 512
NOPS = 40
NBUF = 4
EPT = NOPS * CHUNK
EPAD = EPT * TILES
PAD = EPAD - E
N_PAD = 10240
RPT = N_PAD // TILES

BN = 1000


def _build_edges(adj0, adj1):
    spread = jnp.arange(PAD, dtype=jnp.int32)
    pad_src = spread % N_PAD
    pad_dst = N + spread % (N_PAD - N)
    src = jnp.concatenate([adj0[0], pad_src, adj1[0] + N_PAD, pad_src])
    dst = jnp.concatenate([adj0[1], pad_dst, adj1[1], pad_dst])
    return (src.reshape(NUM_SC * TILES * NOPS, CHUNK),
            dst.reshape(NUM_SC * TILES * NOPS, CHUNK))


def _sc_scatter_body(zcat, srcm, dstm, out, srcg, dstg, rows, acc,
                     gs0, gs1, gs2, gs3, ss0, ss1, ss2, ss3):
    gsems = [gs0, gs1, gs2, gs3]
    ssems = [ss0, ss1, ss2, ss3]
    cid = lax.axis_index("c")
    sid = lax.axis_index("s")

    zvec = jnp.zeros((16,), jnp.float32)

    def _memset(r, carry):
        rows[0, r, pl.ds(0, 16)] = zvec
        rows[0, r, pl.ds(16, 16)] = zvec
        return carry

    lax.fori_loop(0, RPT, _memset, 0, unroll=False)
    pltpu.sync_copy(rows.at[0, pl.ds(0, RPT)], acc.at[pl.ds(sid * RPT, RPT)])

    row0 = (cid * TILES + sid) * NOPS
    pltpu.sync_copy(srcm.at[pl.ds(row0, NOPS)], srcg)
    pltpu.sync_copy(dstm.at[pl.ds(row0, NOPS)], dstg)

    plsc.subcore_barrier()

    def gather_start(k, b):
        pltpu.async_copy(zcat.at[srcg.at[k]], rows.at[b], gsems[b])

    def gather_wait(k, b):
        pltpu.make_async_copy(zcat.at[srcg.at[k]], rows.at[b],
                              gsems[b]).wait()

    def scatter_start(k, b):
        pltpu.async_copy(rows.at[b], acc.at[dstg.at[k]], ssems[b],
                         add=True)

    def scatter_wait(k, b):
        pltpu.make_async_copy(rows.at[b], acc.at[dstg.at[k]],
                              ssems[b]).wait()

    for b in range(NBUF):
        gather_start(b, b)

    def body(i, carry):
        k = i * NBUF
        for b in range(NBUF):
            gather_wait(k + b, b)
            scatter_start(k + b, b)
        for b in range(NBUF):
            @pl.when(k + b + NBUF < NOPS)
            def _():
                scatter_wait(k + b, b)
                gather_start(k + b + NBUF, b)
        return carry

    lax.fori_loop(0, NOPS // NBUF, body, 0, unroll=False)
    for b in range(NBUF):
        scatter_wait(NOPS - NBUF + b, b)

    plsc.subcore_barrier()

    pltpu.sync_copy(acc.at[pl.ds(sid * RPT, RPT)],
                    out.at[pl.ds(cid * N_PAD + sid * RPT, RPT)])


_sc_scatter = functools.partial(
    pl.kernel,
    out_type=jax.ShapeDtypeStruct((NUM_SC * N_PAD, HEAD), jnp.float32),
    mesh=plsc.VectorSubcoreMesh(core_axis_name="c", subcore_axis_name="s"),
    scratch_types=[
        pltpu.VMEM((NOPS, CHUNK), jnp.int32),
        pltpu.VMEM((NOPS, CHUNK), jnp.int32),
        pltpu.VMEM((NBUF, CHUNK, HEAD), jnp.float32),
        pltpu.VMEM_SHARED((N_PAD, HEAD), jnp.float32),
        pltpu.SemaphoreType.DMA,
        pltpu.SemaphoreType.DMA,
        pltpu.SemaphoreType.DMA,
        pltpu.SemaphoreType.DMA,
        pltpu.SemaphoreType.DMA,
        pltpu.SemaphoreType.DMA,
        pltpu.SemaphoreType.DMA,
        pltpu.SemaphoreType.DMA,
    ],
    compiler_params=pltpu.CompilerParams(use_tc_tiling_on_sc=False),
)(_sc_scatter_body)


def _mm_in_body(x_ref, w_ref, o_ref):
    z = jnp.dot(x_ref[...], w_ref[...], preferred_element_type=jnp.float32)
    o_ref[0] = z[:, :HEAD]
    o_ref[1] = z[:, HEAD:]


def _layer0_matmul(x, wcat):
    return pl.pallas_call(
        _mm_in_body,
        grid=(N // BN,),
        in_specs=[
            pl.BlockSpec((BN, D_IN), lambda i: (i, 0)),
            pl.BlockSpec((D_IN, 2 * HEAD), lambda i: (0, 0)),
        ],
        out_specs=pl.BlockSpec((2, BN, HEAD), lambda i: (0, i, 0)),
        out_shape=jax.ShapeDtypeStruct((2, N_PAD, HEAD), jnp.float32),
    )(x, wcat)


def _mid_body(agg_ref, b0_ref, b1_ref, w0_ref, w1_ref, h_ref, z_ref):
    h0 = jax.nn.relu(agg_ref[0] + b0_ref[...])
    h1 = jax.nn.relu(agg_ref[1] + b1_ref[...])
    hb = jnp.concatenate([h0, h1], axis=1)
    h_ref[...] = hb
    z_ref[0] = jnp.dot(hb, w0_ref[...], preferred_element_type=jnp.float32)
    z_ref[1] = jnp.dot(hb, w1_ref[...], preferred_element_type=jnp.float32)


def _layer_mid(agg, b00, b01, W10, W11):
    return pl.pallas_call(
        _mid_body,
        grid=(N // BN,),
        in_specs=[
            pl.BlockSpec((2, BN, HEAD), lambda i: (0, i, 0)),
            pl.BlockSpec((1, HEAD), lambda i: (0, 0)),
            pl.BlockSpec((1, HEAD), lambda i: (0, 0)),
            pl.BlockSpec((2 * HEAD, HEAD), lambda i: (0, 0)),
            pl.BlockSpec((2 * HEAD, HEAD), lambda i: (0, 0)),
        ],
        out_specs=[
            pl.BlockSpec((BN, 2 * HEAD), lambda i: (i, 0)),
            pl.BlockSpec((2, BN, HEAD), lambda i: (0, i, 0)),
        ],
        out_shape=[
            jax.ShapeDtypeStruct((N, 2 * HEAD), jnp.float32),
            jax.ShapeDtypeStruct((2, N_PAD, HEAD), jnp.float32),
        ],
    )(agg, b00, b01, W10, W11)


def _final_body(agg_ref, h1_ref, b0_ref, b1_ref, o_ref):
    h0 = jax.nn.relu(agg_ref[0] + b0_ref[...])
    h1 = jax.nn.relu(agg_ref[1] + b1_ref[...])
    o_ref[...] = jnp.concatenate([h1_ref[...], h0, h1], axis=1)


def _layer_final(agg, h1, b10, b11):
    return pl.pallas_call(
        _final_body,
        grid=(N // BN,),
        in_specs=[
            pl.BlockSpec((2, BN, HEAD), lambda i: (0, i, 0)),
            pl.BlockSpec((BN, 2 * HEAD), lambda i: (i, 0)),
            pl.BlockSpec((1, HEAD), lambda i: (0, 0)),
            pl.BlockSpec((1, HEAD), lambda i: (0, 0)),
        ],
        out_specs=pl.BlockSpec((BN, 4 * HEAD), lambda i: (i, 0)),
        out_shape=jax.ShapeDtypeStruct((N, 4 * HEAD), jnp.float32),
    )(agg, h1, b10, b11)


def kernel(x, adj0, adj1, W00, b00, W01, b01, W10, b10, W11, b11):
    srcm, dstm = _build_edges(adj0, adj1)

    z0 = _layer0_matmul(x, jnp.concatenate([W00, W01], axis=1))
    agg0 = _sc_scatter(z0.reshape(NUM_SC * N_PAD, HEAD), srcm, dstm)
    h1, z1 = _layer_mid(agg0.reshape(NUM_SC, N_PAD, HEAD),
                        b00.reshape(1, HEAD), b01.reshape(1, HEAD), W10, W11)
    agg1 = _sc_scatter(z1.reshape(NUM_SC * N_PAD, HEAD), srcm, dstm)
    return _layer_final(agg1.reshape(NUM_SC, N_PAD, HEAD), h1,
                        b10.reshape(1, HEAD), b11.reshape(1, HEAD))


# speedup vs baseline: 23.7105x; 1.0585x over previous
import functools

import jax
import jax.numpy as jnp
from jax import lax
from jax.experimental import pallas as pl
from jax.experimental.pallas import tpu as pltpu
from jax.experimental.pallas import tpu_sc as plsc

N = 10000
E = 320000
D_IN = 128
HEAD = 32

NUM_SC = 2
TILES = 16
CHUNK = 512
NCHUNKS = E // CHUNK
NOPS = 39
NBUF = 3
N_PAD = 10240
RPT = N_PAD // TILES

BN = 1000


def _sc_scatter_body(z0t, z1t, adj0r, adj1r, out, srcg, dstg, rows, acc,
                     gs0, gs1, gs2, ss0, ss1, ss2):
    gsems = [gs0, gs1, gs2]
    ssems = [ss0, ss1, ss2]
    cid = lax.axis_index("c")
    sid = lax.axis_index("s")

    zvec = jnp.zeros((16,), jnp.float32)

    def _memset(r, carry):
        rows[0, r, pl.ds(0, 16)] = zvec
        rows[0, r, pl.ds(16, 16)] = zvec
        return carry

    lax.fori_loop(0, CHUNK, _memset, 0, unroll=False)
    pltpu.sync_copy(rows.at[0, pl.ds(0, CHUNK)],
                    acc.at[pl.ds(sid * RPT, CHUNK)])
    pltpu.sync_copy(rows.at[0, pl.ds(0, RPT - CHUNK)],
                    acc.at[pl.ds(sid * RPT + CHUNK, RPT - CHUNK)])

    def stage(adjr):
        pltpu.sync_copy(adjr.at[0, pl.ds(sid * NOPS, NOPS)],
                        srcg.at[pl.ds(0, NOPS)])
        pltpu.sync_copy(adjr.at[1, pl.ds(sid * NOPS, NOPS)],
                        dstg.at[pl.ds(0, NOPS)])

        @pl.when(sid == 0)
        def _():
            pltpu.sync_copy(adjr.at[0, pl.ds(NCHUNKS - 1, 1)],
                            srcg.at[pl.ds(NOPS, 1)])
            pltpu.sync_copy(adjr.at[1, pl.ds(NCHUNKS - 1, 1)],
                            dstg.at[pl.ds(NOPS, 1)])

    @pl.when(cid == 0)
    def _():
        stage(adj0r)

    @pl.when(cid == 1)
    def _():
        stage(adj1r)

    plsc.subcore_barrier()

    def pipeline(ztab):
        def gather_start(k, b):
            pltpu.async_copy(ztab.at[srcg.at[k]], rows.at[b], gsems[b])

        def gather_wait(k, b):
            pltpu.make_async_copy(ztab.at[srcg.at[k]], rows.at[b],
                                  gsems[b]).wait()

        def scatter_start(k, b):
            pltpu.async_copy(rows.at[b], acc.at[dstg.at[k]], ssems[b],
                             add=True)

        def scatter_wait(k, b):
            pltpu.make_async_copy(rows.at[b], acc.at[dstg.at[k]],
                                  ssems[b]).wait()

        for b in range(NBUF):
            gather_start(b, b)

        def body(i, carry):
            k = i * NBUF
            for b in range(NBUF):
                gather_wait(k + b, b)
                scatter_start(k + b, b)
            for b in range(NBUF):
                @pl.when(k + b + NBUF < NOPS)
                def _():
                    scatter_wait(k + b, b)
                    gather_start(k + b + NBUF, b)
            return carry

        lax.fori_loop(0, NOPS // NBUF, body, 0, unroll=False)
        for b in range(NBUF):
            scatter_wait(NOPS - NBUF + b, b)

        @pl.when(sid == 0)
        def _():
            gather_start(NOPS, 0)
            gather_wait(NOPS, 0)
            scatter_start(NOPS, 0)
            scatter_wait(NOPS, 0)

    @pl.when(cid == 0)
    def _():
        pipeline(z0t)

    @pl.when(cid == 1)
    def _():
        pipeline(z1t)

    plsc.subcore_barrier()

    pltpu.sync_copy(acc.at[pl.ds(sid * RPT, RPT)],
                    out.at[pl.ds(cid * N_PAD + sid * RPT, RPT)])


_sc_scatter = functools.partial(
    pl.kernel,
    out_type=jax.ShapeDtypeStruct((NUM_SC * N_PAD, HEAD), jnp.float32),
    mesh=plsc.VectorSubcoreMesh(core_axis_name="c", subcore_axis_name="s"),
    scratch_types=[
        pltpu.VMEM((NOPS + 1, CHUNK), jnp.int32),
        pltpu.VMEM((NOPS + 1, CHUNK), jnp.int32),
        pltpu.VMEM((NBUF, CHUNK, HEAD), jnp.float32),
        pltpu.VMEM_SHARED((N_PAD, HEAD), jnp.float32),
        pltpu.SemaphoreType.DMA,
        pltpu.SemaphoreType.DMA,
        pltpu.SemaphoreType.DMA,
        pltpu.SemaphoreType.DMA,
        pltpu.SemaphoreType.DMA,
        pltpu.SemaphoreType.DMA,
    ],
    compiler_params=pltpu.CompilerParams(use_tc_tiling_on_sc=False),
)(_sc_scatter_body)


def _mm_in_body(x_ref, w_ref, o0_ref, o1_ref):
    z = jnp.dot(x_ref[...], w_ref[...], preferred_element_type=jnp.float32)
    o0_ref[...] = z[:, :HEAD]
    o1_ref[...] = z[:, HEAD:]


def _layer0_matmul(x, wcat):
    return pl.pallas_call(
        _mm_in_body,
        grid=(N // BN,),
        in_specs=[
            pl.BlockSpec((BN, D_IN), lambda i: (i, 0)),
            pl.BlockSpec((D_IN, 2 * HEAD), lambda i: (0, 0)),
        ],
        out_specs=[
            pl.BlockSpec((BN, HEAD), lambda i: (i, 0)),
            pl.BlockSpec((BN, HEAD), lambda i: (i, 0)),
        ],
        out_shape=[
            jax.ShapeDtypeStruct((N_PAD, HEAD), jnp.float32),
            jax.ShapeDtypeStruct((N_PAD, HEAD), jnp.float32),
        ],
    )(x, wcat)


def _mid_body(agg_ref, b0_ref, b1_ref, w0_ref, w1_ref, h_ref, z0_ref,
              z1_ref):
    h0 = jax.nn.relu(agg_ref[0] + b0_ref[...])
    h1 = jax.nn.relu(agg_ref[1] + b1_ref[...])
    hb = jnp.concatenate([h0, h1], axis=1)
    h_ref[...] = hb
    z0_ref[...] = jnp.dot(hb, w0_ref[...], preferred_element_type=jnp.float32)
    z1_ref[...] = jnp.dot(hb, w1_ref[...], preferred_element_type=jnp.float32)


def _layer_mid(agg, b00, b01, W10, W11):
    return pl.pallas_call(
        _mid_body,
        grid=(N // BN,),
        in_specs=[
            pl.BlockSpec((2, BN, HEAD), lambda i: (0, i, 0)),
            pl.BlockSpec((1, HEAD), lambda i: (0, 0)),
            pl.BlockSpec((1, HEAD), lambda i: (0, 0)),
            pl.BlockSpec((2 * HEAD, HEAD), lambda i: (0, 0)),
            pl.BlockSpec((2 * HEAD, HEAD), lambda i: (0, 0)),
        ],
        out_specs=[
            pl.BlockSpec((BN, 2 * HEAD), lambda i: (i, 0)),
            pl.BlockSpec((BN, HEAD), lambda i: (i, 0)),
            pl.BlockSpec((BN, HEAD), lambda i: (i, 0)),
        ],
        out_shape=[
            jax.ShapeDtypeStruct((N, 2 * HEAD), jnp.float32),
            jax.ShapeDtypeStruct((N_PAD, HEAD), jnp.float32),
            jax.ShapeDtypeStruct((N_PAD, HEAD), jnp.float32),
        ],
    )(agg, b00, b01, W10, W11)


def _final_body(agg_ref, h1_ref, b0_ref, b1_ref, o_ref):
    h0 = jax.nn.relu(agg_ref[0] + b0_ref[...])
    h1 = jax.nn.relu(agg_ref[1] + b1_ref[...])
    o_ref[...] = jnp.concatenate([h1_ref[...], h0, h1], axis=1)


def _layer_final(agg, h1, b10, b11):
    return pl.pallas_call(
        _final_body,
        grid=(N // BN,),
        in_specs=[
            pl.BlockSpec((2, BN, HEAD), lambda i: (0, i, 0)),
            pl.BlockSpec((BN, 2 * HEAD), lambda i: (i, 0)),
            pl.BlockSpec((1, HEAD), lambda i: (0, 0)),
            pl.BlockSpec((1, HEAD), lambda i: (0, 0)),
        ],
        out_specs=pl.BlockSpec((BN, 4 * HEAD), lambda i: (i, 0)),
        out_shape=jax.ShapeDtypeStruct((N, 4 * HEAD), jnp.float32),
    )(agg, h1, b10, b11)


def kernel(x, adj0, adj1, W00, b00, W01, b01, W10, b10, W11, b11):
    adj0r = adj0.reshape(2, NCHUNKS, CHUNK)
    adj1r = adj1.reshape(2, NCHUNKS, CHUNK)

    z00, z01 = _layer0_matmul(x, jnp.concatenate([W00, W01], axis=1))
    agg0 = _sc_scatter(z00, z01, adj0r, adj1r)
    h1, z10, z11 = _layer_mid(agg0.reshape(NUM_SC, N_PAD, HEAD),
                              b00.reshape(1, HEAD), b01.reshape(1, HEAD),
                              W10, W11)
    agg1 = _sc_scatter(z10, z11, adj0r, adj1r)
    return _layer_final(agg1.reshape(NUM_SC, N_PAD, HEAD), h1,
                        b10.reshape(1, HEAD), b11.reshape(1, HEAD))


# speedup vs baseline: 24.5672x; 1.0361x over previous
import functools

import jax
import jax.numpy as jnp
from jax import lax
from jax.experimental import pallas as pl
from jax.experimental.pallas import tpu as pltpu
from jax.experimental.pallas import tpu_sc as plsc

N = 10000
E = 320000
D_IN = 128
HEAD = 32

NUM_SC = 2
TILES = 16
CHUNK = 512
NCHUNKS = E // CHUNK
NOPS = 39
NBUF = 4
NMAIN = 36
N_PAD = 10240
RPT = N_PAD // TILES

BN = 1000


def _sc_scatter_body(z0t, z1t, adj0r, adj1r, out, srcg, dstg, rows, acc,
                     gs0, gs1, gs2, gs3, ss0, ss1, ss2, ss3):
    gsems = [gs0, gs1, gs2, gs3]
    ssems = [ss0, ss1, ss2, ss3]
    cid = lax.axis_index("c")
    sid = lax.axis_index("s")

    zvec = jnp.zeros((16,), jnp.float32)

    def _memset(r, carry):
        rows[0, r, pl.ds(0, 16)] = zvec
        rows[0, r, pl.ds(16, 16)] = zvec
        return carry

    lax.fori_loop(0, CHUNK, _memset, 0, unroll=False)
    pltpu.sync_copy(rows.at[0, pl.ds(0, CHUNK)],
                    acc.at[pl.ds(sid * RPT, CHUNK)])
    pltpu.sync_copy(rows.at[0, pl.ds(0, RPT - CHUNK)],
                    acc.at[pl.ds(sid * RPT + CHUNK, RPT - CHUNK)])

    def stage(adjr):
        pltpu.sync_copy(adjr.at[0, pl.ds(sid * NOPS, NOPS)],
                        srcg.at[pl.ds(0, NOPS)])
        pltpu.sync_copy(adjr.at[1, pl.ds(sid * NOPS, NOPS)],
                        dstg.at[pl.ds(0, NOPS)])

        @pl.when(sid == 0)
        def _():
            pltpu.sync_copy(adjr.at[0, pl.ds(NCHUNKS - 1, 1)],
                            srcg.at[pl.ds(NOPS, 1)])
            pltpu.sync_copy(adjr.at[1, pl.ds(NCHUNKS - 1, 1)],
                            dstg.at[pl.ds(NOPS, 1)])

    @pl.when(cid == 0)
    def _():
        stage(adj0r)

    @pl.when(cid == 1)
    def _():
        stage(adj1r)

    plsc.subcore_barrier()

    def pipeline(ztab):
        def gather_start(k, b):
            pltpu.async_copy(ztab.at[srcg.at[k]], rows.at[b], gsems[b])

        def gather_wait(k, b):
            pltpu.make_async_copy(ztab.at[srcg.at[k]], rows.at[b],
                                  gsems[b]).wait()

        def scatter_start(k, b):
            pltpu.async_copy(rows.at[b], acc.at[dstg.at[k]], ssems[b],
                             add=True)

        def scatter_wait(k, b):
            pltpu.make_async_copy(rows.at[b], acc.at[dstg.at[k]],
                                  ssems[b]).wait()

        for b in range(NBUF):
            gather_start(b, b)

        def body(i, carry):
            k = i * NBUF
            for b in range(NBUF):
                gather_wait(k + b, b)
                scatter_start(k + b, b)
            for b in range(NBUF):
                @pl.when(k + b + NBUF < NOPS)
                def _():
                    scatter_wait(k + b, b)
                    gather_start(k + b + NBUF, b)
            return carry

        lax.fori_loop(0, NMAIN // NBUF, body, 0, unroll=False)
        for b in range(NOPS - NMAIN):
            gather_wait(NMAIN + b, b)
            scatter_start(NMAIN + b, b)
        scatter_wait(NMAIN - 1, NBUF - 1)
        for b in range(NOPS - NMAIN):
            scatter_wait(NMAIN + b, b)

        @pl.when(sid == 0)
        def _():
            gather_start(NOPS, NBUF - 1)
            gather_wait(NOPS, NBUF - 1)
            scatter_start(NOPS, NBUF - 1)
            scatter_wait(NOPS, NBUF - 1)

    @pl.when(cid == 0)
    def _():
        pipeline(z0t)

    @pl.when(cid == 1)
    def _():
        pipeline(z1t)

    plsc.subcore_barrier()

    pltpu.sync_copy(acc.at[pl.ds(sid * RPT, RPT)],
                    out.at[pl.ds(cid * N_PAD + sid * RPT, RPT)])


_sc_scatter = functools.partial(
    pl.kernel,
    out_type=jax.ShapeDtypeStruct((NUM_SC * N_PAD, HEAD), jnp.float32),
    mesh=plsc.VectorSubcoreMesh(core_axis_name="c", subcore_axis_name="s"),
    scratch_types=[
        pltpu.VMEM((NOPS + 1, CHUNK), jnp.int32),
        pltpu.VMEM((NOPS + 1, CHUNK), jnp.int32),
        pltpu.VMEM((NBUF, CHUNK, HEAD), jnp.float32),
        pltpu.VMEM_SHARED((N_PAD, HEAD), jnp.float32),
        pltpu.SemaphoreType.DMA,
        pltpu.SemaphoreType.DMA,
        pltpu.SemaphoreType.DMA,
        pltpu.SemaphoreType.DMA,
        pltpu.SemaphoreType.DMA,
        pltpu.SemaphoreType.DMA,
        pltpu.SemaphoreType.DMA,
        pltpu.SemaphoreType.DMA,
    ],
    compiler_params=pltpu.CompilerParams(use_tc_tiling_on_sc=False),
)(_sc_scatter_body)


def _mm_in_body(x_ref, w_ref, o0_ref, o1_ref):
    z = jnp.dot(x_ref[...], w_ref[...], preferred_element_type=jnp.float32)
    o0_ref[...] = z[:, :HEAD]
    o1_ref[...] = z[:, HEAD:]


def _layer0_matmul(x, wcat):
    return pl.pallas_call(
        _mm_in_body,
        grid=(N // BN,),
        in_specs=[
            pl.BlockSpec((BN, D_IN), lambda i: (i, 0)),
            pl.BlockSpec((D_IN, 2 * HEAD), lambda i: (0, 0)),
        ],
        out_specs=[
            pl.BlockSpec((BN, HEAD), lambda i: (i, 0)),
            pl.BlockSpec((BN, HEAD), lambda i: (i, 0)),
        ],
        out_shape=[
            jax.ShapeDtypeStruct((N_PAD, HEAD), jnp.float32),
            jax.ShapeDtypeStruct((N_PAD, HEAD), jnp.float32),
        ],
    )(x, wcat)


def _mid_body(agg_ref, b0_ref, b1_ref, w0_ref, w1_ref, h_ref, z0_ref,
              z1_ref):
    h0 = jax.nn.relu(agg_ref[0] + b0_ref[...])
    h1 = jax.nn.relu(agg_ref[1] + b1_ref[...])
    hb = jnp.concatenate([h0, h1], axis=1)
    h_ref[...] = hb
    z0_ref[...] = jnp.dot(hb, w0_ref[...], preferred_element_type=jnp.float32)
    z1_ref[...] = jnp.dot(hb, w1_ref[...], preferred_element_type=jnp.float32)


def _layer_mid(agg, b00, b01, W10, W11):
    return pl.pallas_call(
        _mid_body,
        grid=(N // BN,),
        in_specs=[
            pl.BlockSpec((2, BN, HEAD), lambda i: (0, i, 0)),
            pl.BlockSpec((1, HEAD), lambda i: (0, 0)),
            pl.BlockSpec((1, HEAD), lambda i: (0, 0)),
            pl.BlockSpec((2 * HEAD, HEAD), lambda i: (0, 0)),
            pl.BlockSpec((2 * HEAD, HEAD), lambda i: (0, 0)),
        ],
        out_specs=[
            pl.BlockSpec((BN, 2 * HEAD), lambda i: (i, 0)),
            pl.BlockSpec((BN, HEAD), lambda i: (i, 0)),
            pl.BlockSpec((BN, HEAD), lambda i: (i, 0)),
        ],
        out_shape=[
            jax.ShapeDtypeStruct((N, 2 * HEAD), jnp.float32),
            jax.ShapeDtypeStruct((N_PAD, HEAD), jnp.float32),
            jax.ShapeDtypeStruct((N_PAD, HEAD), jnp.float32),
        ],
    )(agg, b00, b01, W10, W11)


def _final_body(agg_ref, h1_ref, b0_ref, b1_ref, o_ref):
    h0 = jax.nn.relu(agg_ref[0] + b0_ref[...])
    h1 = jax.nn.relu(agg_ref[1] + b1_ref[...])
    o_ref[...] = jnp.concatenate([h1_ref[...], h0, h1], axis=1)


def _layer_final(agg, h1, b10, b11):
    return pl.pallas_call(
        _final_body,
        grid=(N // BN,),
        in_specs=[
            pl.BlockSpec((2, BN, HEAD), lambda i: (0, i, 0)),
            pl.BlockSpec((BN, 2 * HEAD), lambda i: (i, 0)),
            pl.BlockSpec((1, HEAD), lambda i: (0, 0)),
            pl.BlockSpec((1, HEAD), lambda i: (0, 0)),
        ],
        out_specs=pl.BlockSpec((BN, 4 * HEAD), lambda i: (i, 0)),
        out_shape=jax.ShapeDtypeStruct((N, 4 * HEAD), jnp.float32),
    )(agg, h1, b10, b11)


def kernel(x, adj0, adj1, W00, b00, W01, b01, W10, b10, W11, b11):
    adj0r = adj0.reshape(2, NCHUNKS, CHUNK)
    adj1r = adj1.reshape(2, NCHUNKS, CHUNK)

    z00, z01 = _layer0_matmul(x, jnp.concatenate([W00, W01], axis=1))
    agg0 = _sc_scatter(z00, z01, adj0r, adj1r)
    h1, z10, z11 = _layer_mid(agg0.reshape(NUM_SC, N_PAD, HEAD),
                              b00.reshape(1, HEAD), b01.reshape(1, HEAD),
                              W10, W11)
    agg1 = _sc_scatter(z10, z11, adj0r, adj1r)
    return _layer_final(agg1.reshape(NUM_SC, N_PAD, HEAD), h1,
                        b10.reshape(1, HEAD), b11.reshape(1, HEAD))
